# baseline (speedup 1.0000x reference)
import functools

import jax
import jax.numpy as jnp
from jax import lax
from jax.experimental import pallas as pl
from jax.experimental.pallas import tpu as pltpu
from jax.experimental.pallas import tpu_sc as plsc

DIM = 128
MAX_DIS = 1000
B = 100000
C = 128
NW = 32
SPAN = 3200
NCH = SPAN // C
NBUF = 6
LOOKAHEAD = 4

_mesh = plsc.VectorSubcoreMesh(core_axis_name="c", subcore_axis_name="s")


@functools.partial(
    pl.kernel,
    mesh=_mesh,
    out_type=jax.ShapeDtypeStruct((B, DIM), jnp.float32),
    scratch_types=[
        pltpu.VMEM((SPAN,), jnp.int32),
        pltpu.VMEM((NBUF, C, DIM), jnp.float32),
        pltpu.SemaphoreType.DMA((NBUF,)),
        pltpu.SemaphoreType.DMA((NBUF,)),
    ],
)
def _gather_kernel(d_hbm, embed_hbm, out_hbm, idx_v, rows_v, gsem, wsem):
    wid = lax.axis_index("s") * 2 + lax.axis_index("c")
    base_w = jnp.minimum(wid * SPAN, B - SPAN)

    pltpu.sync_copy(d_hbm.at[pl.ds(base_w, SPAN)], idx_v)

    def clamp_body(j, carry):
        sl = pl.ds(j * 16, 16)
        idx_v[sl] = jnp.minimum(jnp.maximum(idx_v[sl], 0), MAX_DIS)
        return carry

    lax.fori_loop(0, SPAN // 16, clamp_body, None)

    def gather_start(k, b):
        pltpu.make_async_copy(
            embed_hbm.at[idx_v.at[pl.ds(k * C, C)]], rows_v.at[b], gsem.at[b]
        ).start()

    def gather_wait(b):
        pltpu.make_async_copy(
            embed_hbm.at[idx_v.at[pl.ds(0, C)]], rows_v.at[b], gsem.at[b]
        ).wait()

    def write_start(k, b):
        pltpu.make_async_copy(
            rows_v.at[b], out_hbm.at[pl.ds(base_w + k * C, C)], wsem.at[b]
        ).start()

    def write_wait(b):
        pltpu.make_async_copy(
            rows_v.at[b], out_hbm.at[pl.ds(0, C)], wsem.at[b]
        ).wait()

    for k in range(LOOKAHEAD):
        gather_start(k, k % NBUF)

    def chunk_body(k, carry):
        kf = k + LOOKAHEAD

        @pl.when(kf < NCH)
        def _():
            bf = lax.rem(kf, NBUF)

            @pl.when(kf >= NBUF)
            def _():
                write_wait(bf)

            gather_start(kf, bf)

        b = lax.rem(k, NBUF)
        gather_wait(b)
        write_start(k, b)
        return carry

    lax.fori_loop(0, NCH, chunk_body, None)

    for b in range(NBUF):
        write_wait(b)


def kernel(d, embed_d):
    return _gather_kernel(d, embed_d)


# speedup vs baseline: 29.3863x; 29.3863x over previous
import functools

import jax
import jax.numpy as jnp
from jax import lax
from jax.experimental import pallas as pl
from jax.experimental.pallas import tpu as pltpu
from jax.experimental.pallas import tpu_sc as plsc

DIM = 128
MAX_DIS = 1000
B = 100000
C = 128
NW = 32
SPAN = 3200
NCH = SPAN // C
NBUF = 6
LOOKAHEAD = 4

_mesh = plsc.VectorSubcoreMesh(core_axis_name="c", subcore_axis_name="s")


@functools.partial(
    pl.kernel,
    mesh=_mesh,
    out_type=jax.ShapeDtypeStruct((B, DIM), jnp.float32),
    scratch_types=[
        pltpu.VMEM((SPAN,), jnp.int32),
        pltpu.VMEM((NBUF, C, DIM), jnp.float32),
        pltpu.VMEM_SHARED((MAX_DIS + 1, DIM), jnp.float32),
        pltpu.SemaphoreType.DMA((NBUF,)),
        pltpu.SemaphoreType.DMA((NBUF,)),
    ],
)
def _gather_kernel(d_hbm, embed_hbm, out_hbm, idx_v, rows_v, table_sh,
                   gsem, wsem):
    sid = lax.axis_index("s")
    wid = sid * 2 + lax.axis_index("c")
    base_w = jnp.minimum(wid * SPAN, B - SPAN)

    @pl.when(sid == 0)
    def _():
        pltpu.sync_copy(embed_hbm, table_sh)

    plsc.subcore_barrier()

    pltpu.sync_copy(d_hbm.at[pl.ds(base_w, SPAN)], idx_v)

    def clamp_body(j, carry):
        sl = pl.ds(j * 16, 16)
        idx_v[sl] = jnp.minimum(jnp.maximum(idx_v[sl], 0), MAX_DIS)
        return carry

    lax.fori_loop(0, SPAN // 16, clamp_body, None)

    def gather_start(k, b):
        pltpu.make_async_copy(
            table_sh.at[idx_v.at[pl.ds(k * C, C)]], rows_v.at[b], gsem.at[b]
        ).start()

    def gather_wait(b):
        pltpu.make_async_copy(
            table_sh.at[idx_v.at[pl.ds(0, C)]], rows_v.at[b], gsem.at[b]
        ).wait()

    def write_start(k, b):
        pltpu.make_async_copy(
            rows_v.at[b], out_hbm.at[pl.ds(base_w + k * C, C)], wsem.at[b]
        ).start()

    def write_wait(b):
        pltpu.make_async_copy(
            rows_v.at[b], out_hbm.at[pl.ds(0, C)], wsem.at[b]
        ).wait()

    for k in range(LOOKAHEAD):
        gather_start(k, k % NBUF)

    def chunk_body(k, carry):
        kf = k + LOOKAHEAD

        @pl.when(kf < NCH)
        def _():
            bf = lax.rem(kf, NBUF)

            @pl.when(kf >= NBUF)
            def _():
                write_wait(bf)

            gather_start(kf, bf)

        b = lax.rem(k, NBUF)
        gather_wait(b)
        write_start(k, b)
        return carry

    lax.fori_loop(0, NCH, chunk_body, None)

    for b in range(NBUF):
        write_wait(b)


def kernel(d, embed_d):
    return _gather_kernel(d, embed_d)


# speedup vs baseline: 30.4956x; 1.0378x over previous
import functools

import jax
import jax.numpy as jnp
from jax import lax
from jax.experimental import pallas as pl
from jax.experimental.pallas import tpu as pltpu
from jax.experimental.pallas import tpu_sc as plsc

DIM = 128
MAX_DIS = 1000
B = 100000
C = 128
NW = 32
SPAN = 3200
NCH = SPAN // C
NBUF = 6
LOOKAHEAD = 4

_mesh = plsc.VectorSubcoreMesh(core_axis_name="c", subcore_axis_name="s")


@functools.partial(
    pl.kernel,
    mesh=_mesh,
    out_type=jax.ShapeDtypeStruct((B, DIM), jnp.float32),
    scratch_types=[
        pltpu.VMEM((SPAN,), jnp.int32),
        pltpu.VMEM((NBUF, C, DIM), jnp.float32),
        pltpu.VMEM_SHARED((MAX_DIS + 1, DIM), jnp.float32),
        pltpu.SemaphoreType.DMA((NBUF,)),
        pltpu.SemaphoreType.DMA((NBUF,)),
    ],
)
def _gather_kernel(d_hbm, embed_hbm, out_hbm, idx_v, rows_v, table_sh,
                   gsem, wsem):
    sid = lax.axis_index("s")
    wid = sid * 2 + lax.axis_index("c")
    base_w = jnp.minimum(wid * SPAN, B - SPAN)

    @pl.when(sid == 0)
    def _():
        pltpu.sync_copy(embed_hbm, table_sh)

    pltpu.sync_copy(d_hbm.at[pl.ds(base_w, SPAN)], idx_v)

    def clamp_body(j, carry):
        sl = pl.ds(j * 16, 16)
        idx_v[sl] = jnp.minimum(jnp.maximum(idx_v[sl], 0), MAX_DIS)
        return carry

    lax.fori_loop(0, SPAN // 16, clamp_body, None)

    plsc.subcore_barrier()

    def gather_start(k, b):
        pltpu.make_async_copy(
            table_sh.at[idx_v.at[pl.ds(k * C, C)]], rows_v.at[b], gsem.at[b]
        ).start()

    def gather_wait(b):
        pltpu.make_async_copy(
            table_sh.at[idx_v.at[pl.ds(0, C)]], rows_v.at[b], gsem.at[b]
        ).wait()

    def write_start(k, b):
        pltpu.make_async_copy(
            rows_v.at[b], out_hbm.at[pl.ds(base_w + k * C, C)], wsem.at[b]
        ).start()

    def write_wait(b):
        pltpu.make_async_copy(
            rows_v.at[b], out_hbm.at[pl.ds(0, C)], wsem.at[b]
        ).wait()

    for k in range(LOOKAHEAD):
        gather_start(k, k % NBUF)

    def chunk_body(k, carry):
        kf = k + LOOKAHEAD

        @pl.when(kf < NCH)
        def _():
            bf = lax.rem(kf, NBUF)

            @pl.when(kf >= NBUF)
            def _():
                write_wait(bf)

            gather_start(kf, bf)

        b = lax.rem(k, NBUF)
        gather_wait(b)
        write_start(k, b)
        return carry

    lax.fori_loop(0, NCH, chunk_body, None)

    for b in range(NBUF):
        write_wait(b)


def kernel(d, embed_d):
    return _gather_kernel(d, embed_d)
